# baseline (speedup 1.0000x reference)
import jax
import jax.numpy as jnp
from jax import lax
from jax.experimental import pallas as pl
from jax.experimental.pallas import tpu as pltpu
from jax.experimental.pallas import tpu_sc as plsc

N, C, D = 4096, 200, 128
NT = N * C
NCORE, NSUB = 2, 16
NW = NCORE * NSUB
TPW = NT // NW
CH = 256
NCHUNK = TPW // CH
GB = 128
NGB = CH // GB
L = 16
DB = D // L


def _body(gid_hbm, mod_hbm, expr_hbm, tt_hbm, wg_hbm, wmext_hbm, wex_hbm,
          out_hbm,
          gbuf0, gbuf1, gid0, gid1, mod0, mod1, tt0, tt1, expr0, expr1,
          m0f0, m0f1, em20, em21, mxb0, mxb1, wm_v, wex_v,
          gsem, osem0, osem1, ssem0, ssem1):
    slots = [
        dict(gbuf=gbuf0, gid=gid0, mod=mod0, tt=tt0, expr=expr0, m0f=m0f0,
             em2=em20, mxb=mxb0, osem=osem0, ssem=ssem0),
        dict(gbuf=gbuf1, gid=gid1, mod=mod1, tt=tt1, expr=expr1, m0f=m0f1,
             em2=em21, mxb=mxb1, osem=osem1, ssem=ssem1),
    ]
    wid = lax.axis_index("s") * NCORE + lax.axis_index("c")
    base0 = wid * TPW

    pltpu.sync_copy(wmext_hbm, wm_v)
    pltpu.sync_copy(wex_hbm, wex_v)
    wvecs = [wex_v[pl.ds(L * j, L)] for j in range(DB)]
    iota = lax.iota(jnp.int32, L)

    def chunk_base(i):
        return base0 + lax.rem(jnp.int32(i + NCHUNK), NCHUNK) * CH

    def scalar_copies(i, S):
        b = chunk_base(i)
        return [
            (gid_hbm.at[pl.ds(b, CH)], S["gid"]),
            (mod_hbm.at[pl.ds(b, CH)], S["mod"]),
            (tt_hbm.at[pl.ds(b, CH)], S["tt"]),
            (expr_hbm.at[pl.ds(b, CH)], S["expr"]),
        ]

    def fire_scalars(i, S):
        for src, dst in scalar_copies(i, S):
            pltpu.async_copy(src, dst, S["ssem"])

    def wait_scalars(i, S):
        for src, dst in scalar_copies(i, S):
            pltpu.make_async_copy(src, dst, S["ssem"]).wait()

    def precompute(S):
        for ii in range(CH // L):
            s = pl.ds(ii * L, L)
            tt = S["tt"][s]
            S["m0f"][s] = (tt & 1).astype(jnp.float32)
            S["em2"][s] = S["expr"][s] * ((tt >> 2) & 1).astype(jnp.float32)
            S["mxb"][s] = (((tt >> 1) & 1) << 10) | (S["mod"][s] << 7)

    def gather_copies(S):
        return [
            (wg_hbm.at[S["gid"].at[pl.ds(j * GB, GB)]],
             S["gbuf"].at[pl.ds(j * GB, GB)])
            for j in range(NGB)
        ]

    def fire_gathers(S):
        for src, dst in gather_copies(S):
            pltpu.async_copy(src, dst, gsem)

    def wait_gathers(S):
        for src, dst in gather_copies(S):
            pltpu.make_async_copy(src, dst, gsem).wait()

    def token_pass(S):
        m0f, em2, mxb, gbuf = S["m0f"], S["em2"], S["mxb"], S["gbuf"]

        @plsc.parallel_loop(0, CH, 1, unroll=16)
        def tok(t):
            vt = jnp.full((L,), t, jnp.int32)
            vm0 = plsc.load_gather(m0f, [vt])
            vem2 = plsc.load_gather(em2, [vt])
            mi = plsc.load_gather(mxb, [vt]) + iota
            for j in range(DB):
                vmod = plsc.load_gather(wm_v, [mi + (j * L)])
                vg = gbuf[t, pl.ds(j * L, L)]
                gbuf[t, pl.ds(j * L, L)] = vg * vm0 + vmod + wvecs[j] * vem2

    def fire_out(i, S):
        pltpu.async_copy(S["gbuf"], out_hbm.at[pl.ds(chunk_base(i), CH)],
                         S["osem"])

    def wait_out(i, S):
        pltpu.make_async_copy(S["gbuf"], out_hbm.at[pl.ds(chunk_base(i), CH)],
                              S["osem"]).wait()

    fire_scalars(0, slots[0])
    wait_scalars(0, slots[0])
    precompute(slots[0])
    fire_gathers(slots[0])
    fire_scalars(1, slots[1])

    def pair(k, _):
        for b in (0, 1):
            S, O = slots[b], slots[1 - b]
            i = 2 * k + b
            wait_gathers(S)
            token_pass(S)
            fire_out(i, S)
            fire_scalars(i + 2, S)
            if b == 0:
                @pl.when(k > 0)
                def _():
                    wait_out(i - 1, O)
            else:
                wait_out(i - 1, O)
            wait_scalars(i + 1, O)
            precompute(O)
            fire_gathers(O)
        return 0

    lax.fori_loop(0, NCHUNK // 2, pair, 0)

    wait_gathers(slots[0])
    wait_scalars(1, slots[1])
    wait_out(NCHUNK - 1, slots[1])


_sc_call = pl.kernel(
    _body,
    out_type=jax.ShapeDtypeStruct((NT, D), jnp.float32),
    mesh=plsc.VectorSubcoreMesh(core_axis_name="c", subcore_axis_name="s"),
    compiler_params=pltpu.CompilerParams(needs_layout_passes=False),
    scratch_types=[
        pltpu.VMEM((CH, D), jnp.float32),
        pltpu.VMEM((CH, D), jnp.float32),
        pltpu.VMEM((CH,), jnp.int32),
        pltpu.VMEM((CH,), jnp.int32),
        pltpu.VMEM((CH,), jnp.int32),
        pltpu.VMEM((CH,), jnp.int32),
        pltpu.VMEM((CH,), jnp.int32),
        pltpu.VMEM((CH,), jnp.int32),
        pltpu.VMEM((CH,), jnp.float32),
        pltpu.VMEM((CH,), jnp.float32),
        pltpu.VMEM((CH,), jnp.float32),
        pltpu.VMEM((CH,), jnp.float32),
        pltpu.VMEM((CH,), jnp.float32),
        pltpu.VMEM((CH,), jnp.float32),
        pltpu.VMEM((CH,), jnp.int32),
        pltpu.VMEM((CH,), jnp.int32),
        pltpu.VMEM((16 * D,), jnp.float32),
        pltpu.VMEM((D,), jnp.float32),
        pltpu.SemaphoreType.DMA,
        pltpu.SemaphoreType.DMA,
        pltpu.SemaphoreType.DMA,
        pltpu.SemaphoreType.DMA,
        pltpu.SemaphoreType.DMA,
    ],
)


@jax.jit
def kernel(gene_id, modality, expression, token_type_nc, W_gene, W_modality,
           w_expr):
    gid = gene_id.reshape(NT).astype(jnp.int32)
    mod = modality.reshape(NT).astype(jnp.int32)
    tt = token_type_nc.reshape(NT).astype(jnp.int32)
    expr = expression.reshape(NT).astype(jnp.float32)
    wmext = jnp.concatenate(
        [jnp.zeros((8, D), jnp.float32), W_modality.astype(jnp.float32)],
        axis=0).reshape(-1)
    out = _sc_call(gid, mod, expr, tt, W_gene, wmext, w_expr)
    return out.reshape(N, C, D)


# speedup vs baseline: 2.0157x; 2.0157x over previous
import jax
import jax.numpy as jnp
from jax import lax
from jax.experimental import pallas as pl
from jax.experimental.pallas import tpu as pltpu
from jax.experimental.pallas import tpu_sc as plsc

N, C, D = 4096, 200, 128
NT = N * C
NCORE, NSUB = 2, 16
NW = NCORE * NSUB
TPW = NT // NW
CH = 256
NCHUNK = TPW // CH
GB = 128
NGB = CH // GB
L = 16
DB = D // L
NSLOT = 3
NTRIPLE = NCHUNK // NSLOT


def _body(gid_hbm, mod_hbm, expr_hbm, tt_hbm, wg_hbm, wmext_hbm, wex_hbm,
          out_hbm, *scr):
    names = ("gbuf", "gid", "mod", "tt", "expr", "m0f", "em2", "mxb",
             "gsem", "osem", "ssem")
    slots = [dict(zip(names, scr[s * len(names):(s + 1) * len(names)]))
             for s in range(NSLOT)]
    wm_v, wex_v = scr[NSLOT * len(names):]

    wid = lax.axis_index("s") * NCORE + lax.axis_index("c")
    base0 = wid * TPW

    pltpu.sync_copy(wmext_hbm, wm_v)
    pltpu.sync_copy(wex_hbm, wex_v)
    wvecs = [wex_v[pl.ds(L * j, L)] for j in range(DB)]
    iota = lax.iota(jnp.int32, L)

    def chunk_base(i):
        return base0 + i * CH

    def scalar_copies(i, S):
        b = chunk_base(i)
        return [
            (gid_hbm.at[pl.ds(b, CH)], S["gid"]),
            (mod_hbm.at[pl.ds(b, CH)], S["mod"]),
            (tt_hbm.at[pl.ds(b, CH)], S["tt"]),
            (expr_hbm.at[pl.ds(b, CH)], S["expr"]),
        ]

    def fire_scalars(i, S):
        for src, dst in scalar_copies(i, S):
            pltpu.async_copy(src, dst, S["ssem"])

    def wait_scalars(i, S):
        for src, dst in scalar_copies(i, S):
            pltpu.make_async_copy(src, dst, S["ssem"]).wait()

    def precompute(S):
        for ii in range(CH // L):
            s = pl.ds(ii * L, L)
            tt = S["tt"][s]
            S["m0f"][s] = (tt & 1).astype(jnp.float32)
            S["em2"][s] = S["expr"][s] * ((tt >> 2) & 1).astype(jnp.float32)
            S["mxb"][s] = (((tt >> 1) & 1) << 10) | (S["mod"][s] << 7)

    def gather_copies(S):
        return [
            (wg_hbm.at[S["gid"].at[pl.ds(j * GB, GB)]],
             S["gbuf"].at[pl.ds(j * GB, GB)])
            for j in range(NGB)
        ]

    def fire_gathers(S):
        for src, dst in gather_copies(S):
            pltpu.async_copy(src, dst, S["gsem"])

    def wait_gathers(S):
        for src, dst in gather_copies(S):
            pltpu.make_async_copy(src, dst, S["gsem"]).wait()

    def token_pass(S):
        m0f, em2, mxb, gbuf = S["m0f"], S["em2"], S["mxb"], S["gbuf"]

        @plsc.parallel_loop(0, CH, 1, unroll=8)
        def tok(t):
            vt = jnp.full((L,), t, jnp.int32)
            vm0 = plsc.load_gather(m0f, [vt])
            vem2 = plsc.load_gather(em2, [vt])
            mi = plsc.load_gather(mxb, [vt]) + iota
            for j in range(DB):
                vmod = plsc.load_gather(wm_v, [mi + (j * L)])
                vg = gbuf[t, pl.ds(j * L, L)]
                gbuf[t, pl.ds(j * L, L)] = vg * vm0 + vmod + wvecs[j] * vem2

    def fire_out(i, S):
        pltpu.async_copy(S["gbuf"], out_hbm.at[pl.ds(chunk_base(i), CH)],
                         S["osem"])

    def wait_out(i, S):
        pltpu.make_async_copy(S["gbuf"], out_hbm.at[pl.ds(chunk_base(i), CH)],
                              S["osem"]).wait()

    def step(i, b, static_tail=False):
        S = slots[b]
        Sp = slots[(b + 2) % NSLOT]

        wait_gathers(S)
        if static_tail:
            if NCHUNK > 3:
                pass
        else:
            @pl.when(i + 3 < NCHUNK)
            def _():
                fire_scalars(i + 3, S)
        token_pass(S)
        fire_out(i, S)
        if static_tail:
            wait_out(i - 1, Sp)
        else:
            @pl.when(i > 0)
            def _():
                wait_out(i - 1, Sp)

            @pl.when(i + 2 < NCHUNK)
            def _():
                wait_scalars(i + 2, Sp)
                precompute(Sp)
                fire_gathers(Sp)

    for s in range(NSLOT):
        fire_scalars(s, slots[s])
    for s in range(2):
        wait_scalars(s, slots[s])
        precompute(slots[s])
        fire_gathers(slots[s])

    def triple(k, _):
        for b in range(NSLOT):
            step(NSLOT * k + b, b)
        return 0

    lax.fori_loop(0, NTRIPLE, triple, 0)
    step(NCHUNK - 1, (NCHUNK - 1) % NSLOT, static_tail=True)
    wait_out(NCHUNK - 1, slots[(NCHUNK - 1) % NSLOT])


_slot_scratch = [
    pltpu.VMEM((CH, D), jnp.float32),
    pltpu.VMEM((CH,), jnp.int32),
    pltpu.VMEM((CH,), jnp.int32),
    pltpu.VMEM((CH,), jnp.int32),
    pltpu.VMEM((CH,), jnp.float32),
    pltpu.VMEM((CH,), jnp.float32),
    pltpu.VMEM((CH,), jnp.float32),
    pltpu.VMEM((CH,), jnp.int32),
    pltpu.SemaphoreType.DMA,
    pltpu.SemaphoreType.DMA,
    pltpu.SemaphoreType.DMA,
]

_sc_call = pl.kernel(
    _body,
    out_type=jax.ShapeDtypeStruct((NT, D), jnp.float32),
    mesh=plsc.VectorSubcoreMesh(core_axis_name="c", subcore_axis_name="s"),
    compiler_params=pltpu.CompilerParams(needs_layout_passes=False),
    scratch_types=(
        _slot_scratch * NSLOT
        + [
            pltpu.VMEM((16 * D,), jnp.float32),
            pltpu.VMEM((D,), jnp.float32),
        ]
    ),
)


@jax.jit
def kernel(gene_id, modality, expression, token_type_nc, W_gene, W_modality,
           w_expr):
    gid = gene_id.reshape(NT).astype(jnp.int32)
    mod = modality.reshape(NT).astype(jnp.int32)
    tt = token_type_nc.reshape(NT).astype(jnp.int32)
    expr = expression.reshape(NT).astype(jnp.float32)
    wmext = jnp.concatenate(
        [jnp.zeros((8, D), jnp.float32), W_modality.astype(jnp.float32)],
        axis=0).reshape(-1)
    out = _sc_call(gid, mod, expr, tt, W_gene, wmext, w_expr)
    return out.reshape(N, C, D)
